# baseline (speedup 1.0000x reference)
import functools

import jax
import jax.numpy as jnp
from jax.experimental import pallas as pl
from jax.experimental.pallas import tpu as pltpu

B = 16384
VISION_DIM = 768
PROMPT_DIM = 768
NUM_PROMPTS = 1024
TOP_K = 8

BM = 256


def _body(vf_ref, wt_ref, b_ref, pool_ref, out_ref):
    s0 = (
        jnp.dot(vf_ref[...], wt_ref[...], preferred_element_type=jnp.float32)
        + b_ref[...]
    )
    m1 = jnp.max(s0, axis=1, keepdims=True)
    s = s0
    m = m1
    for _ in range(TOP_K - 1):
        s = jnp.where(s == m, -jnp.inf, s)
        m = jnp.max(s, axis=1, keepdims=True)
    e = jnp.where(s0 >= m, jnp.exp(s0 - m1), 0.0)
    z = jnp.sum(e, axis=1, keepdims=True)
    w = (e / z).astype(jnp.bfloat16)
    out_ref[...] = jnp.dot(w, pool_ref[...], preferred_element_type=jnp.float32)


@jax.jit
def kernel(vision_features, W, b, prompt_pool):
    wt = W.T
    b2 = b.reshape(1, NUM_PROMPTS)
    pool_bf = prompt_pool.astype(jnp.bfloat16)
    grid = (B // BM,)
    return pl.pallas_call(
        _body,
        grid=grid,
        in_specs=[
            pl.BlockSpec((BM, VISION_DIM), lambda i: (i, 0)),
            pl.BlockSpec((VISION_DIM, NUM_PROMPTS), lambda i: (0, 0)),
            pl.BlockSpec((1, NUM_PROMPTS), lambda i: (0, 0)),
            pl.BlockSpec((NUM_PROMPTS, PROMPT_DIM), lambda i: (0, 0)),
        ],
        out_specs=pl.BlockSpec((BM, PROMPT_DIM), lambda i: (i, 0)),
        out_shape=jax.ShapeDtypeStruct((B, PROMPT_DIM), jnp.float32),
        compiler_params=pltpu.CompilerParams(
            dimension_semantics=("parallel",),
        ),
    )(vision_features, wt, b2, pool_bf)


# speedup vs baseline: 1.1291x; 1.1291x over previous
import functools

import jax
import jax.numpy as jnp
from jax.experimental import pallas as pl
from jax.experimental.pallas import tpu as pltpu

B = 16384
VISION_DIM = 768
PROMPT_DIM = 768
NUM_PROMPTS = 1024
TOP_K = 8

BM = 512


def _body(vf_ref, wt_ref, b_ref, pool_ref, out_ref):
    s0 = (
        jnp.dot(vf_ref[...], wt_ref[...], preferred_element_type=jnp.float32)
        + b_ref[...]
    )
    m1 = jnp.max(s0, axis=1, keepdims=True)
    s = s0
    m = m1
    for _ in range(TOP_K - 1):
        s = jnp.where(s == m, -jnp.inf, s)
        m = jnp.max(s, axis=1, keepdims=True)
    e = jnp.where(s0 >= m, jnp.exp(s0 - m1), 0.0)
    z = jnp.sum(e, axis=1, keepdims=True)
    w = e / z
    out_ref[...] = jnp.dot(w, pool_ref[...], preferred_element_type=jnp.float32)


@jax.jit
def kernel(vision_features, W, b, prompt_pool):
    wt = W.T
    b2 = b.reshape(1, NUM_PROMPTS)
    grid = (B // BM,)
    return pl.pallas_call(
        _body,
        grid=grid,
        in_specs=[
            pl.BlockSpec((BM, VISION_DIM), lambda i: (i, 0)),
            pl.BlockSpec((VISION_DIM, NUM_PROMPTS), lambda i: (0, 0)),
            pl.BlockSpec((1, NUM_PROMPTS), lambda i: (0, 0)),
            pl.BlockSpec((NUM_PROMPTS, PROMPT_DIM), lambda i: (0, 0)),
        ],
        out_specs=pl.BlockSpec((BM, PROMPT_DIM), lambda i: (i, 0)),
        out_shape=jax.ShapeDtypeStruct((B, PROMPT_DIM), jnp.float32),
        compiler_params=pltpu.CompilerParams(
            dimension_semantics=("parallel",),
        ),
    )(vision_features, wt, b2, prompt_pool)


# speedup vs baseline: 1.1819x; 1.0468x over previous
import functools

import jax
import jax.numpy as jnp
from jax.experimental import pallas as pl
from jax.experimental.pallas import tpu as pltpu

B = 16384
VISION_DIM = 768
PROMPT_DIM = 768
NUM_PROMPTS = 1024
TOP_K = 8

BM = 1024


def _body(vf_ref, wt_ref, b_ref, pool_ref, out_ref):
    s0 = (
        jnp.dot(vf_ref[...], wt_ref[...], preferred_element_type=jnp.float32)
        + b_ref[...]
    )
    m1 = jnp.max(s0, axis=1, keepdims=True)
    s = s0
    m = m1
    for _ in range(TOP_K - 1):
        s = jnp.where(s == m, -jnp.inf, s)
        m = jnp.max(s, axis=1, keepdims=True)
    e = jnp.where(s0 >= m, jnp.exp(s0 - m1), 0.0)
    z = jnp.sum(e, axis=1, keepdims=True)
    w = e / z
    out_ref[...] = jnp.dot(w, pool_ref[...], preferred_element_type=jnp.float32)


@jax.jit
def kernel(vision_features, W, b, prompt_pool):
    wt = W.T
    b2 = b.reshape(1, NUM_PROMPTS)
    grid = (B // BM,)
    return pl.pallas_call(
        _body,
        grid=grid,
        in_specs=[
            pl.BlockSpec((BM, VISION_DIM), lambda i: (i, 0)),
            pl.BlockSpec((VISION_DIM, NUM_PROMPTS), lambda i: (0, 0)),
            pl.BlockSpec((1, NUM_PROMPTS), lambda i: (0, 0)),
            pl.BlockSpec((NUM_PROMPTS, PROMPT_DIM), lambda i: (0, 0)),
        ],
        out_specs=pl.BlockSpec((BM, PROMPT_DIM), lambda i: (i, 0)),
        out_shape=jax.ShapeDtypeStruct((B, PROMPT_DIM), jnp.float32),
        compiler_params=pltpu.CompilerParams(
            dimension_semantics=("parallel",),
        ),
    )(vision_features, wt, b2, prompt_pool)


# speedup vs baseline: 1.1884x; 1.0056x over previous
import functools

import jax
import jax.numpy as jnp
from jax.experimental import pallas as pl
from jax.experimental.pallas import tpu as pltpu

B = 16384
VISION_DIM = 768
PROMPT_DIM = 768
NUM_PROMPTS = 1024
TOP_K = 8

BM = 2048


def _body(vf_ref, wt_ref, b_ref, pool_ref, out_ref):
    s0 = (
        jnp.dot(vf_ref[...], wt_ref[...], preferred_element_type=jnp.float32)
        + b_ref[...]
    )
    m1 = jnp.max(s0, axis=1, keepdims=True)
    s = s0
    m = m1
    for _ in range(TOP_K - 1):
        s = jnp.where(s == m, -jnp.inf, s)
        m = jnp.max(s, axis=1, keepdims=True)
    e = jnp.where(s0 >= m, jnp.exp(s0 - m1), 0.0)
    z = jnp.sum(e, axis=1, keepdims=True)
    w = e / z
    out_ref[...] = jnp.dot(w, pool_ref[...], preferred_element_type=jnp.float32)


@jax.jit
def kernel(vision_features, W, b, prompt_pool):
    wt = W.T
    b2 = b.reshape(1, NUM_PROMPTS)
    grid = (B // BM,)
    return pl.pallas_call(
        _body,
        grid=grid,
        in_specs=[
            pl.BlockSpec((BM, VISION_DIM), lambda i: (i, 0)),
            pl.BlockSpec((VISION_DIM, NUM_PROMPTS), lambda i: (0, 0)),
            pl.BlockSpec((1, NUM_PROMPTS), lambda i: (0, 0)),
            pl.BlockSpec((NUM_PROMPTS, PROMPT_DIM), lambda i: (0, 0)),
        ],
        out_specs=pl.BlockSpec((BM, PROMPT_DIM), lambda i: (i, 0)),
        out_shape=jax.ShapeDtypeStruct((B, PROMPT_DIM), jnp.float32),
        compiler_params=pltpu.CompilerParams(
            dimension_semantics=("parallel",),
        ),
    )(vision_features, wt, b2, prompt_pool)
